# baseline (speedup 1.0000x reference)
import jax
import jax.numpy as jnp
from jax import lax
from jax.experimental import pallas as pl
from jax.experimental.pallas import tpu as pltpu
from jax.experimental.pallas import tpu_sc as plsc

NL = 8
N = 4096
K = 16
B = 256
NC, NS, LANES = 2, 16, 16
NPT = N // NS
CH = 8
NCHUNK = NPT // CH
ROWS = CH * K
BH = B // 2
WH = BH // 2
GW = WH // LANES
EPL = N * K


def _tree_sum(terms):
    while len(terms) > 1:
        terms = [terms[i] + terms[i + 1]
                 for i in range(0, len(terms) - 1, 2)] + \
                (terms[-1:] if len(terms) % 2 else [])
    return terms[0]


def _body(xw_hbm, src_hbm, w_hbm, bias_hbm, tabA, tabB, out_hbm,
          idx0, idx1, w0, w1, b0, b1, rows0, rows1,
          accw0, accw1, accf0, accf1,
          gsem0, gsem1, ssem, psem):
    half = lax.axis_index("c")
    tile = lax.axis_index("s")
    nbase = tile * NPT
    rbase = half * N + nbase

    idx_v = (idx0, idx1)
    w_v = (w0, w1)
    bias_v = (b0, b1)
    rows = (rows0, rows1)
    accw = (accw0, accw1)
    accf = (accf0, accf1)
    gsem = (gsem0, gsem1)

    def stage_args(l):
        pb = l % 2
        return (
            (src_hbm.at[half, pl.ds(l * EPL + nbase * K, NPT * K)],
             idx_v[pb], psem),
            (w_hbm.at[pl.ds(l * EPL + nbase * K, NPT * K)], w_v[pb], psem),
            (bias_hbm.at[pl.ds(l * N + nbase, NPT)], bias_v[pb], psem),
        )

    def gdesc(src_tab, l, c, par):
        pb = l % 2
        return pltpu.make_async_copy(
            src_tab.at[idx_v[pb].at[pl.ds(c * ROWS, ROWS)]],
            rows[par], gsem[par])

    def gstart(src_tab, l, c, par):
        pb = l % 2
        pltpu.async_copy(
            src_tab.at[idx_v[pb].at[pl.ds(c * ROWS, ROWS)]],
            rows[par], gsem[par])

    def sdesc(dst_tab, c, par, final):
        a = accf if final else accw
        return pltpu.make_async_copy(
            a[par], dst_tab.at[pl.ds(rbase + c * CH, CH)], ssem)

    def sstart(dst_tab, c, par, final):
        a = accf if final else accw
        pltpu.async_copy(
            a[par], dst_tab.at[pl.ds(rbase + c * CH, CH)], ssem)

    def compute(l, c, par, final):
        pb = l % 2
        rows_v = rows[par]
        acc_v = accf[par] if final else accw[par]

        @plsc.parallel_loop(0, CH)
        def _neuron(j):
            e0 = c * ROWS + j * K
            wvs = [
                plsc.load_gather(
                    w_v[pb], [jnp.full((LANES,), e0 + k, dtype=jnp.int32)])
                for k in range(K)
            ]
            bv = plsc.load_gather(
                bias_v[pb], [jnp.full((LANES,), c * CH + j, dtype=jnp.int32)])

            @plsc.parallel_loop(0, GW)
            def _grp(g):
                ta, tb = [], []
                for k in range(K):
                    words = rows_v[j * K + k, pl.ds(g * LANES, LANES)]
                    ra, rb = plsc.unpack(
                        plsc.bitcast(words, jnp.bfloat16),
                        format=plsc.PackFormat.INTERLEAVED)
                    ta.append(wvs[k] * ra)
                    tb.append(wvs[k] * rb)
                a = _tree_sum(ta) + bv
                b = _tree_sum(tb) + bv
                if not final:
                    a = a / (1.0 + jnp.exp(-a))
                    b = b / (1.0 + jnp.exp(-b))
                    acc_v[j, pl.ds(g * LANES, LANES)] = plsc.bitcast(
                        plsc.pack(a, b, format=plsc.PackFormat.INTERLEAVED),
                        jnp.int32)
                else:
                    row_i = jnp.full((LANES,), j, dtype=jnp.int32)
                    cols = 32 * g + 2 * lax.iota(jnp.int32, LANES)
                    plsc.store_scatter(acc_v, [row_i, cols], a)
                    plsc.store_scatter(acc_v, [row_i, cols + 1], b)

    srcs = [xw_hbm, tabA, tabB, tabA, tabB, tabA, tabB]
    dsts = [tabA, tabB, tabA, tabB, tabA, tabB, out_hbm]

    for s, d, sem in stage_args(0):
        pltpu.async_copy(s, d, sem)
    for s, d, sem in stage_args(0):
        pltpu.make_async_copy(s, d, sem).wait()

    for l in range(NL - 1):
        final = l == NL - 2
        src_tab, dst_tab = srcs[l], dsts[l]

        if l + 1 < NL - 1:
            for s, d, sem in stage_args(l + 1):
                pltpu.async_copy(s, d, sem)

        gstart(src_tab, l, 0, 0)
        gstart(src_tab, l, 1, 1)
        gdesc(src_tab, l, 0, 0).wait()
        compute(l, 0, 0, final)
        sstart(dst_tab, 0, 0, final)
        gstart(src_tab, l, 2, 0)
        gdesc(src_tab, l, 1, 1).wait()
        compute(l, 1, 1, final)
        sstart(dst_tab, 1, 1, final)
        gstart(src_tab, l, 3, 1)

        def loop_body(i, _, l=l, final=final, src_tab=src_tab,
                      dst_tab=dst_tab):
            c0 = 2 * i
            for par in (0, 1):
                c = c0 + par
                sdesc(dst_tab, c - 2, par, final).wait()
                gdesc(src_tab, l, c, par).wait()
                compute(l, c, par, final)
                sstart(dst_tab, c, par, final)

                @pl.when(c + 2 < NCHUNK)
                def _prefetch(c=c, par=par):
                    gstart(src_tab, l, c + 2, par)
            return _

        lax.fori_loop(1, NCHUNK // 2, loop_body, 0)
        sdesc(dst_tab, NCHUNK - 2, 0, final).wait()
        sdesc(dst_tab, NCHUNK - 1, 1, final).wait()

        if l + 1 < NL - 1:
            for s, d, sem in stage_args(l + 1):
                pltpu.make_async_copy(s, d, sem).wait()
        plsc.subcore_barrier()


_kernel_call = pl.kernel(
    _body,
    out_type=(
        jax.ShapeDtypeStruct((2 * N, 2 * WH), jnp.int32),
        jax.ShapeDtypeStruct((2 * N, 2 * WH), jnp.int32),
        jax.ShapeDtypeStruct((2 * N, BH), jnp.float32),
    ),
    mesh=plsc.VectorSubcoreMesh(
        core_axis_name="c", subcore_axis_name="s",
        num_cores=NC, num_subcores=NS),
    scratch_types=[
        pltpu.VMEM((NPT * K,), jnp.int32),
        pltpu.VMEM((NPT * K,), jnp.int32),
        pltpu.VMEM((NPT * K,), jnp.float32),
        pltpu.VMEM((NPT * K,), jnp.float32),
        pltpu.VMEM((NPT,), jnp.float32),
        pltpu.VMEM((NPT,), jnp.float32),
        pltpu.VMEM((ROWS, 2 * WH), jnp.int32),
        pltpu.VMEM((ROWS, 2 * WH), jnp.int32),
        pltpu.VMEM((CH, 2 * WH), jnp.int32),
        pltpu.VMEM((CH, 2 * WH), jnp.int32),
        pltpu.VMEM((CH, BH), jnp.float32),
        pltpu.VMEM((CH, BH), jnp.float32),
        pltpu.SemaphoreType.DMA,
        pltpu.SemaphoreType.DMA,
        pltpu.SemaphoreType.DMA,
        pltpu.SemaphoreType.DMA,
    ],
    compiler_params=pltpu.CompilerParams(needs_layout_passes=False),
)


@jax.jit
def kernel(x, edge_src_local, edge_w, bias):
    xt = x.T.astype(jnp.bfloat16)
    xh = lax.bitcast_convert_type(
        xt.reshape(N, 2, WH, 2), jnp.int32)
    xh = xh.transpose(1, 0, 2).reshape(2 * N, WH)
    xw = jnp.concatenate(
        [xh, jnp.zeros((2 * N, WH), jnp.int32)], axis=1)
    src = edge_src_local.reshape(-1)
    src_adj = jnp.stack([src, src + N])
    _, _, out = _kernel_call(
        xw, src_adj, edge_w.reshape(-1), bias.reshape(-1))
    return out.reshape(2, N, BH).transpose(0, 2, 1).reshape(B, N)


# speedup vs baseline: 1.2717x; 1.2717x over previous
import functools

import jax
import jax.numpy as jnp
from jax import lax
from jax.experimental import pallas as pl
from jax.experimental.pallas import tpu as pltpu
from jax.experimental.pallas import tpu_sc as plsc

NL = 8
N = 4096
K = 16
B = 256
NC, NS, LANES = 2, 16, 16
NW = NC * NS
NPW = N // NW
CH = 8
NCHUNK = NPW // CH
ROWS = CH * K
W2 = B // 2
GW = W2 // LANES


def _tree_sum(terms):
    while len(terms) > 1:
        terms = [terms[i] + terms[i + 1]
                 for i in range(0, len(terms) - 1, 2)] + \
                (terms[-1:] if len(terms) % 2 else [])
    return terms[0]


def _layer_body(apply_act, f32_out, vals_hbm, src_hbm, w_hbm, bias_hbm,
                out_hbm, idx_v, w_v, bias_v, rows0, rows1, acc0, acc1,
                gsem0, gsem1, ssem):
    wid = lax.axis_index("s") * NC + lax.axis_index("c")
    base = wid * NPW
    ebase = base * K

    rows = (rows0, rows1)
    acc = (acc0, acc1)
    gsem = (gsem0, gsem1)

    def gather(c):
        return pltpu.async_copy(
            vals_hbm.at[idx_v.at[pl.ds(c * ROWS, ROWS)]], rows[c % 2],
            gsem[c % 2])

    pltpu.sync_copy(src_hbm.at[pl.ds(ebase, NPW * K)], idx_v)
    first = gather(0)
    pltpu.sync_copy(w_hbm.at[pl.ds(ebase, NPW * K)], w_v)
    pltpu.sync_copy(bias_hbm.at[pl.ds(base, NPW)], bias_v)

    def compute(c):
        rows_v, acc_v = rows[c % 2], acc[c % 2]

        @plsc.parallel_loop(0, CH)
        def _neuron(j):
            e0 = c * ROWS + j * K
            w_row = w_v[pl.ds(e0, K)]
            wbs = []
            dnums = lax.GatherDimensionNumbers(
                offset_dims=(), collapsed_slice_dims=(0,),
                start_index_map=(0,))
            for k in range(K):
                ws = lax.gather(
                    w_row, jnp.full((LANES, 1), k, dtype=jnp.int32),
                    dnums, slice_sizes=(1,),
                    mode=lax.GatherScatterMode.PROMISE_IN_BOUNDS)
                wbs.append(plsc.pack(
                    ws, ws, format=plsc.PackFormat.INTERLEAVED))
            bv = plsc.load_gather(
                bias_v, [jnp.full((LANES,), c * CH + j, dtype=jnp.int32)])

            @plsc.parallel_loop(0, GW, unroll=2)
            def _grp(g):
                ps = []
                for k in range(K):
                    words = rows_v[j * K + k, pl.ds(g * LANES, LANES)]
                    ps.append(plsc.bitcast(words, jnp.bfloat16) * wbs[k])
                qs = [ps[i] + ps[i + 1] for i in range(0, K, 2)]
                ta, tb = [], []
                for q in qs:
                    ra, rb = plsc.unpack(
                        q, format=plsc.PackFormat.INTERLEAVED)
                    ta.append(ra)
                    tb.append(rb)
                a = _tree_sum(ta) + bv
                b = _tree_sum(tb) + bv
                if apply_act:
                    a = a / (1.0 + jnp.exp(-a))
                    b = b / (1.0 + jnp.exp(-b))
                if f32_out:
                    row_i = jnp.full((LANES,), j, dtype=jnp.int32)
                    cols = 32 * g + 2 * lax.iota(jnp.int32, LANES)
                    plsc.store_scatter(acc_v, [row_i, cols], a)
                    plsc.store_scatter(acc_v, [row_i, cols + 1], b)
                else:
                    acc_v[j, pl.ds(g * LANES, LANES)] = plsc.bitcast(
                        plsc.pack(a, b, format=plsc.PackFormat.INTERLEAVED),
                        jnp.int32)

    ghandles = {0: first}
    shandles = []
    for c in range(NCHUNK):
        ghandles[c].wait()
        if c + 1 < NCHUNK:
            ghandles[c + 1] = gather(c + 1)
        if c >= 2:
            shandles[c - 2].wait()
        compute(c)
        shandles.append(pltpu.async_copy(
            acc[c % 2], out_hbm.at[pl.ds(base + c * CH, CH)], ssem))
    for h in shandles[-2:]:
        h.wait()


def _make_layer(apply_act, f32_out):
    out_w = B if f32_out else W2
    out_t = jnp.float32 if f32_out else jnp.int32
    return pl.kernel(
        functools.partial(_layer_body, apply_act, f32_out),
        out_type=jax.ShapeDtypeStruct((N, out_w), out_t),
        mesh=plsc.VectorSubcoreMesh(
            core_axis_name="c", subcore_axis_name="s",
            num_cores=NC, num_subcores=NS),
        scratch_types=[
            pltpu.VMEM((NPW * K,), jnp.int32),
            pltpu.VMEM((NPW * K,), jnp.float32),
            pltpu.VMEM((NPW,), jnp.float32),
            pltpu.VMEM((ROWS, W2), jnp.int32),
            pltpu.VMEM((ROWS, W2), jnp.int32),
            pltpu.VMEM((CH, out_w), out_t),
            pltpu.VMEM((CH, out_w), out_t),
            pltpu.SemaphoreType.DMA,
            pltpu.SemaphoreType.DMA,
            pltpu.SemaphoreType.DMA,
        ],
        compiler_params=pltpu.CompilerParams(needs_layout_passes=False),
    )


@jax.jit
def kernel(x, edge_src_local, edge_w, bias):
    xt = x.T.astype(jnp.bfloat16)
    vals = lax.bitcast_convert_type(
        xt.reshape(N, W2, 2), jnp.int32)
    hidden = _make_layer(True, False)
    final = _make_layer(False, True)
    for l in range(NL - 1):
        fn = hidden if l < NL - 2 else final
        vals = fn(vals, edge_src_local[l].reshape(-1),
                  edge_w[l].reshape(-1), bias[l])
    return vals.T


# speedup vs baseline: 1.4867x; 1.1691x over previous
import jax
import jax.numpy as jnp
from jax import lax
from jax.experimental import pallas as pl
from jax.experimental.pallas import tpu as pltpu
from jax.experimental.pallas import tpu_sc as plsc

NL = 8
NLM1 = NL - 1
N = 4096
K = 16
B = 256
NC, NS, LANES = 2, 16, 16
NW = NC * NS
NPW = N // NW
CH = 8
NCHUNK = NPW // CH
ROWS = CH * K
W2 = B // 2
GW = W2 // LANES
EPL = N * K


def _tree_sum(terms):
    while len(terms) > 1:
        terms = [terms[i] + terms[i + 1]
                 for i in range(0, len(terms) - 1, 2)] + \
                (terms[-1:] if len(terms) % 2 else [])
    return terms[0]


def _body(xw_hbm, src_hbm, w_hbm, bias_hbm, tabA, tabB, out_hbm, flags,
          idx0, idx1, w0, w1, b0, b1, rows0, rows1,
          accw0, accw1, accf0, accf1, zbuf, obuf, fbuf,
          gsem0, gsem1, ssem, psem, fsem):
    scid = lax.axis_index("c")
    s = lax.axis_index("s")
    wid = s * NC + scid
    base = wid * NPW
    other = 1 - scid

    idx_v = (idx0, idx1)
    w_v = (w0, w1)
    bias_v = (b0, b1)
    rows = (rows0, rows1)
    accw = (accw0, accw1)
    accf = (accf0, accf1)
    gsem = (gsem0, gsem1)

    obuf[...] = jnp.full((LANES,), 1, dtype=jnp.int32)

    @pl.when(s == 0)
    def _zero_flags():
        for i in range(NLM1):
            zbuf[i, :] = jnp.zeros((LANES,), dtype=jnp.int32)
        pltpu.async_copy(zbuf, flags.at[scid], fsem).wait()

    def stage_args(l):
        pb = l % 2
        return (
            (src_hbm.at[pl.ds(l * EPL + base * K, NPW * K)], idx_v[pb], psem),
            (w_hbm.at[pl.ds(l * EPL + base * K, NPW * K)], w_v[pb], psem),
            (bias_hbm.at[pl.ds(l * N + base, NPW)], bias_v[pb], psem),
        )

    def gstart(src_tab, l, c, par):
        pb = l % 2
        pltpu.async_copy(
            src_tab.at[idx_v[pb].at[pl.ds(c * ROWS, ROWS)]],
            rows[par], gsem[par])

    def gdesc(src_tab, l, c, par):
        pb = l % 2
        return pltpu.make_async_copy(
            src_tab.at[idx_v[pb].at[pl.ds(c * ROWS, ROWS)]],
            rows[par], gsem[par])

    def sdesc(dst_tab, c, par, final):
        a = accf if final else accw
        return pltpu.make_async_copy(
            a[par], dst_tab.at[pl.ds(base + c * CH, CH)], ssem)

    def sstart(dst_tab, c, par, final):
        a = accf if final else accw
        pltpu.async_copy(
            a[par], dst_tab.at[pl.ds(base + c * CH, CH)], ssem)

    def compute(l, c, par, final):
        pb = l % 2
        rows_v = rows[par]
        acc_v = accf[par] if final else accw[par]

        @plsc.parallel_loop(0, CH)
        def _neuron(j):
            e0 = c * ROWS + j * K
            w_row = w_v[pb][pl.ds(e0, K)]
            wbs = []
            dnums = lax.GatherDimensionNumbers(
                offset_dims=(), collapsed_slice_dims=(0,),
                start_index_map=(0,))
            for k in range(K):
                ws = lax.gather(
                    w_row, jnp.full((LANES, 1), k, dtype=jnp.int32),
                    dnums, slice_sizes=(1,),
                    mode=lax.GatherScatterMode.PROMISE_IN_BOUNDS)
                wbs.append(plsc.pack(
                    ws, ws, format=plsc.PackFormat.INTERLEAVED))
            bv = plsc.load_gather(
                bias_v[pb], [jnp.full((LANES,), c * CH + j, dtype=jnp.int32)])

            @plsc.parallel_loop(0, GW, unroll=2)
            def _grp(g):
                ps = []
                for k in range(K):
                    words = rows_v[j * K + k, pl.ds(g * LANES, LANES)]
                    ps.append(plsc.bitcast(words, jnp.bfloat16) * wbs[k])
                qs = [ps[i] + ps[i + 1] for i in range(0, K, 2)]
                ta, tb = [], []
                for q in qs:
                    ra, rb = plsc.unpack(
                        q, format=plsc.PackFormat.INTERLEAVED)
                    ta.append(ra)
                    tb.append(rb)
                a = _tree_sum(ta) + bv
                b = _tree_sum(tb) + bv
                if not final:
                    a = a / (1.0 + jnp.exp(-a))
                    b = b / (1.0 + jnp.exp(-b))
                    acc_v[j, pl.ds(g * LANES, LANES)] = plsc.bitcast(
                        plsc.pack(a, b, format=plsc.PackFormat.INTERLEAVED),
                        jnp.int32)
                else:
                    row_i = jnp.full((LANES,), j, dtype=jnp.int32)
                    cols = 32 * g + 2 * lax.iota(jnp.int32, LANES)
                    plsc.store_scatter(acc_v, [row_i, cols], a)
                    plsc.store_scatter(acc_v, [row_i, cols + 1], b)

    srcs = [xw_hbm, tabA, tabB, tabA, tabB, tabA, tabB]
    dsts = [tabA, tabB, tabA, tabB, tabA, tabB, out_hbm]

    for sr, d, sem in stage_args(0):
        pltpu.async_copy(sr, d, sem)
    for sr, d, sem in stage_args(0):
        pltpu.make_async_copy(sr, d, sem).wait()

    for l in range(NLM1):
        final = l == NLM1 - 1
        src_tab, dst_tab = srcs[l], dsts[l]

        if l + 1 < NLM1:
            for sr, d, sem in stage_args(l + 1):
                pltpu.async_copy(sr, d, sem)

        gstart(src_tab, l, 0, 0)
        gstart(src_tab, l, 1, 1)
        gdesc(src_tab, l, 0, 0).wait()
        compute(l, 0, 0, final)
        sstart(dst_tab, 0, 0, final)
        gstart(src_tab, l, 2, 0)
        gdesc(src_tab, l, 1, 1).wait()
        compute(l, 1, 1, final)
        sstart(dst_tab, 1, 1, final)
        gstart(src_tab, l, 3, 1)

        def loop_body(i, _, l=l, final=final, src_tab=src_tab,
                      dst_tab=dst_tab):
            c0 = 2 * i
            for par in (0, 1):
                c = c0 + par
                sdesc(dst_tab, c - 2, par, final).wait()
                gdesc(src_tab, l, c, par).wait()
                compute(l, c, par, final)
                sstart(dst_tab, c, par, final)

                @pl.when(c + 2 < NCHUNK)
                def _prefetch(c=c, par=par):
                    gstart(src_tab, l, c + 2, par)
            return _

        lax.fori_loop(1, NCHUNK // 2, loop_body, 0)
        sdesc(dst_tab, NCHUNK - 2, 0, final).wait()
        sdesc(dst_tab, NCHUNK - 1, 1, final).wait()

        if l + 1 < NLM1:
            for sr, d, sem in stage_args(l + 1):
                pltpu.make_async_copy(sr, d, sem).wait()

        plsc.subcore_barrier()
        if not final:
            @pl.when(s == 0)
            def _publish(l=l):
                pltpu.async_copy(obuf, flags.at[scid, l], fsem).wait()

            def poll_body(t, l=l):
                pltpu.async_copy(flags.at[other, l], fbuf, fsem).wait()
                return jnp.sum(fbuf[...])

            lax.while_loop(lambda t: t < LANES, poll_body, jnp.int32(0))


_kernel_call = pl.kernel(
    _body,
    out_type=(
        jax.ShapeDtypeStruct((N, W2), jnp.int32),
        jax.ShapeDtypeStruct((N, W2), jnp.int32),
        jax.ShapeDtypeStruct((N, B), jnp.float32),
        jax.ShapeDtypeStruct((NC, NLM1, LANES), jnp.int32),
    ),
    mesh=plsc.VectorSubcoreMesh(
        core_axis_name="c", subcore_axis_name="s",
        num_cores=NC, num_subcores=NS),
    scratch_types=[
        pltpu.VMEM((NPW * K,), jnp.int32),
        pltpu.VMEM((NPW * K,), jnp.int32),
        pltpu.VMEM((NPW * K,), jnp.float32),
        pltpu.VMEM((NPW * K,), jnp.float32),
        pltpu.VMEM((NPW,), jnp.float32),
        pltpu.VMEM((NPW,), jnp.float32),
        pltpu.VMEM((ROWS, W2), jnp.int32),
        pltpu.VMEM((ROWS, W2), jnp.int32),
        pltpu.VMEM((CH, W2), jnp.int32),
        pltpu.VMEM((CH, W2), jnp.int32),
        pltpu.VMEM((CH, B), jnp.float32),
        pltpu.VMEM((CH, B), jnp.float32),
        pltpu.VMEM((NLM1, LANES), jnp.int32),
        pltpu.VMEM((LANES,), jnp.int32),
        pltpu.VMEM((LANES,), jnp.int32),
        pltpu.SemaphoreType.DMA,
        pltpu.SemaphoreType.DMA,
        pltpu.SemaphoreType.DMA,
        pltpu.SemaphoreType.DMA,
        pltpu.SemaphoreType.DMA,
    ],
    compiler_params=pltpu.CompilerParams(needs_layout_passes=False),
)


@jax.jit
def kernel(x, edge_src_local, edge_w, bias):
    xt = x.T.astype(jnp.bfloat16)
    xw = lax.bitcast_convert_type(
        xt.reshape(N, W2, 2), jnp.int32)
    _, _, out, _ = _kernel_call(
        xw, edge_src_local.reshape(-1), edge_w.reshape(-1),
        bias.reshape(-1))
    return out.T


# speedup vs baseline: 1.5378x; 1.0344x over previous
import jax
import jax.numpy as jnp
from jax import lax
from jax.experimental import pallas as pl
from jax.experimental.pallas import tpu as pltpu
from jax.experimental.pallas import tpu_sc as plsc

NL = 8
NLM1 = NL - 1
N = 4096
K = 16
B = 256
NC, NS, LANES = 2, 16, 16
NW = NC * NS
NPW = N // NW
CH = 8
NCHUNK = NPW // CH
ROWS = CH * K
W2 = B // 2
GW = W2 // LANES
EPL = N * K


def _tree_sum(terms):
    while len(terms) > 1:
        terms = [terms[i] + terms[i + 1]
                 for i in range(0, len(terms) - 1, 2)] + \
                (terms[-1:] if len(terms) % 2 else [])
    return terms[0]


def _body(xw_hbm, src_hbm, w_hbm, bias_hbm, tabA, tabB, out_hbm, flags,
          idx0, idx1, w0, w1, b0, b1, rows0, rows1,
          accw0, accw1, accf0, accf1, zbuf, obuf, fbuf,
          gsem0, gsem1, ssem, psem, fsem):
    scid = lax.axis_index("c")
    s = lax.axis_index("s")
    wid = s * NC + scid
    base = wid * NPW
    other = 1 - scid

    idx_v = (idx0, idx1)
    w_v = (w0, w1)
    bias_v = (b0, b1)
    rows = (rows0, rows1)
    accw = (accw0, accw1)
    accf = (accf0, accf1)
    gsem = (gsem0, gsem1)

    obuf[...] = jnp.full((LANES,), 1, dtype=jnp.int32)

    @pl.when(s == 0)
    def _zero_flags():
        for i in range(NLM1):
            zbuf[i, :] = jnp.zeros((LANES,), dtype=jnp.int32)
        pltpu.async_copy(zbuf, flags.at[scid], fsem).wait()

    def stage_args(l):
        pb = l % 2
        return (
            (src_hbm.at[pl.ds(l * EPL + base * K, NPW * K)], idx_v[pb], psem),
            (w_hbm.at[pl.ds(l * EPL + base * K, NPW * K)], w_v[pb], psem),
            (bias_hbm.at[pl.ds(l * N + base, NPW)], bias_v[pb], psem),
        )

    def gstart(src_tab, l, c, par):
        pb = l % 2
        pltpu.async_copy(
            src_tab.at[idx_v[pb].at[pl.ds(c * ROWS, ROWS)]],
            rows[par], gsem[par])

    def gdesc(src_tab, l, c, par):
        pb = l % 2
        return pltpu.make_async_copy(
            src_tab.at[idx_v[pb].at[pl.ds(c * ROWS, ROWS)]],
            rows[par], gsem[par])

    def sdesc(dst_tab, c, par, final):
        a = accf if final else accw
        return pltpu.make_async_copy(
            a[par], dst_tab.at[pl.ds(base + c * CH, CH)], ssem)

    def sstart(dst_tab, c, par, final):
        a = accf if final else accw
        pltpu.async_copy(
            a[par], dst_tab.at[pl.ds(base + c * CH, CH)], ssem)

    def compute(l, c, par, final):
        pb = l % 2
        rows_v = rows[par]
        acc_v = accf[par] if final else accw[par]

        @plsc.parallel_loop(0, CH)
        def _neuron(j):
            e0 = c * ROWS + j * K
            w_row = w_v[pb][pl.ds(e0, K)]
            wbs = []
            dnums = lax.GatherDimensionNumbers(
                offset_dims=(), collapsed_slice_dims=(0,),
                start_index_map=(0,))
            for k in range(K):
                ws = lax.gather(
                    w_row, jnp.full((LANES, 1), k, dtype=jnp.int32),
                    dnums, slice_sizes=(1,),
                    mode=lax.GatherScatterMode.PROMISE_IN_BOUNDS)
                wbs.append(plsc.pack(
                    ws, ws, format=plsc.PackFormat.INTERLEAVED))
            bv = plsc.load_gather(
                bias_v[pb], [jnp.full((LANES,), c * CH + j, dtype=jnp.int32)])

            @plsc.parallel_loop(0, GW, unroll=2)
            def _grp(g):
                ps = []
                for k in range(K):
                    words = rows_v[j * K + k, pl.ds(g * LANES, LANES)]
                    ps.append(plsc.bitcast(words, jnp.bfloat16) * wbs[k])
                qs = [ps[i] + ps[i + 1] for i in range(0, K, 2)]
                rs = [qs[i] + qs[i + 1] for i in range(0, K // 2, 2)]
                ta, tb = [], []
                for q in rs:
                    ra, rb = plsc.unpack(
                        q, format=plsc.PackFormat.INTERLEAVED)
                    ta.append(ra)
                    tb.append(rb)
                a = _tree_sum(ta) + bv
                b = _tree_sum(tb) + bv
                if not final:
                    a = a / (1.0 + jnp.exp(-a))
                    b = b / (1.0 + jnp.exp(-b))
                    acc_v[j, pl.ds(g * LANES, LANES)] = plsc.bitcast(
                        plsc.pack(a, b, format=plsc.PackFormat.INTERLEAVED),
                        jnp.int32)
                else:
                    row_i = jnp.full((LANES,), j, dtype=jnp.int32)
                    cols = 32 * g + 2 * lax.iota(jnp.int32, LANES)
                    plsc.store_scatter(acc_v, [row_i, cols], a)
                    plsc.store_scatter(acc_v, [row_i, cols + 1], b)

    srcs = [xw_hbm, tabA, tabB, tabA, tabB, tabA, tabB]
    dsts = [tabA, tabB, tabA, tabB, tabA, tabB, out_hbm]

    for sr, d, sem in stage_args(0):
        pltpu.async_copy(sr, d, sem)
    for sr, d, sem in stage_args(0):
        pltpu.make_async_copy(sr, d, sem).wait()

    for l in range(NLM1):
        final = l == NLM1 - 1
        src_tab, dst_tab = srcs[l], dsts[l]

        if l + 1 < NLM1:
            for sr, d, sem in stage_args(l + 1):
                pltpu.async_copy(sr, d, sem)

        gstart(src_tab, l, 0, 0)
        gstart(src_tab, l, 1, 1)
        gdesc(src_tab, l, 0, 0).wait()
        compute(l, 0, 0, final)
        sstart(dst_tab, 0, 0, final)
        gstart(src_tab, l, 2, 0)
        gdesc(src_tab, l, 1, 1).wait()
        compute(l, 1, 1, final)
        sstart(dst_tab, 1, 1, final)
        gstart(src_tab, l, 3, 1)

        def loop_body(i, _, l=l, final=final, src_tab=src_tab,
                      dst_tab=dst_tab):
            c0 = 2 * i
            for par in (0, 1):
                c = c0 + par
                sdesc(dst_tab, c - 2, par, final).wait()
                gdesc(src_tab, l, c, par).wait()
                compute(l, c, par, final)
                sstart(dst_tab, c, par, final)

                @pl.when(c + 2 < NCHUNK)
                def _prefetch(c=c, par=par):
                    gstart(src_tab, l, c + 2, par)
            return _

        lax.fori_loop(1, NCHUNK // 2, loop_body, 0)
        sdesc(dst_tab, NCHUNK - 2, 0, final).wait()
        sdesc(dst_tab, NCHUNK - 1, 1, final).wait()

        if l + 1 < NLM1:
            for sr, d, sem in stage_args(l + 1):
                pltpu.make_async_copy(sr, d, sem).wait()

        plsc.subcore_barrier()
        if not final:
            @pl.when(s == 0)
            def _publish(l=l):
                pltpu.async_copy(obuf, flags.at[scid, l], fsem).wait()

            def poll_body(t, l=l):
                pltpu.async_copy(flags.at[other, l], fbuf, fsem).wait()
                return jnp.sum(fbuf[...])

            lax.while_loop(lambda t: t < LANES, poll_body, jnp.int32(0))


_kernel_call = pl.kernel(
    _body,
    out_type=(
        jax.ShapeDtypeStruct((N, W2), jnp.int32),
        jax.ShapeDtypeStruct((N, W2), jnp.int32),
        jax.ShapeDtypeStruct((N, B), jnp.float32),
        jax.ShapeDtypeStruct((NC, NLM1, LANES), jnp.int32),
    ),
    mesh=plsc.VectorSubcoreMesh(
        core_axis_name="c", subcore_axis_name="s",
        num_cores=NC, num_subcores=NS),
    scratch_types=[
        pltpu.VMEM((NPW * K,), jnp.int32),
        pltpu.VMEM((NPW * K,), jnp.int32),
        pltpu.VMEM((NPW * K,), jnp.float32),
        pltpu.VMEM((NPW * K,), jnp.float32),
        pltpu.VMEM((NPW,), jnp.float32),
        pltpu.VMEM((NPW,), jnp.float32),
        pltpu.VMEM((ROWS, W2), jnp.int32),
        pltpu.VMEM((ROWS, W2), jnp.int32),
        pltpu.VMEM((CH, W2), jnp.int32),
        pltpu.VMEM((CH, W2), jnp.int32),
        pltpu.VMEM((CH, B), jnp.float32),
        pltpu.VMEM((CH, B), jnp.float32),
        pltpu.VMEM((NLM1, LANES), jnp.int32),
        pltpu.VMEM((LANES,), jnp.int32),
        pltpu.VMEM((LANES,), jnp.int32),
        pltpu.SemaphoreType.DMA,
        pltpu.SemaphoreType.DMA,
        pltpu.SemaphoreType.DMA,
        pltpu.SemaphoreType.DMA,
        pltpu.SemaphoreType.DMA,
    ],
    compiler_params=pltpu.CompilerParams(needs_layout_passes=False),
)


@jax.jit
def kernel(x, edge_src_local, edge_w, bias):
    xt = x.T.astype(jnp.bfloat16)
    xw = lax.bitcast_convert_type(
        xt.reshape(N, W2, 2), jnp.int32)
    _, _, out, _ = _kernel_call(
        xw, edge_src_local.reshape(-1), edge_w.reshape(-1),
        bias.reshape(-1))
    return out.T
